# baseline (speedup 1.0000x reference)
import functools

import jax
import jax.numpy as jnp
from jax import lax
from jax.experimental import pallas as pl
from jax.experimental.pallas import tpu as pltpu
from jax.experimental.pallas import tpu_sc as plsc

PAD_ID = 999999
D = 64
NC = 2
NS = 16
NW = NC * NS
NBUF = 2


def _gather_body(n_chunks, chunk, idx_hbm, t2_hbm, out_hbm,
                 idx_v, idx2_v, rows, comp, gsems, osems):
    wid = lax.axis_index("s") * NC + lax.axis_index("c")
    bpw = n_chunks * chunk
    base = wid * bpw

    pltpu.sync_copy(idx_hbm.at[wid], idx_v)

    @pl.loop(0, n_chunks)
    def _(g):
        for r0 in range(0, chunk, 16):
            idx2_v[g, pl.ds(r0, 16)] = idx_v[g, pl.ds(r0, 16)] >> 1

    def gather(g, j):
        return pltpu.async_copy(t2_hbm.at[idx2_v.at[g]], rows[j], gsems[j])

    def put_src(j):
        return comp[j]

    def out_slice(g):
        return out_hbm.at[pl.ds(base + g * chunk, chunk)]

    def put(g, j):
        return pltpu.async_copy(put_src(j), out_slice(g), osems[j])

    def wait_gather(g, j):
        pltpu.make_async_copy(t2_hbm.at[idx2_v.at[g]], rows[j],
                              gsems[j]).wait()

    def wait_put(g, j):
        pltpu.make_async_copy(put_src(j), out_slice(g), osems[j]).wait()

    def compact(g, j):
        @pl.loop(0, chunk, step=16)
        def _(r0):
            r_vec = r0 + lax.iota(jnp.int32, 16)
            p_vec = (idx_v[g, pl.ds(r0, 16)] & 1) * 64
            for c in range(D):
                x = plsc.load_gather(rows[j], [r_vec, p_vec + c])
                plsc.store_scatter(comp[j], [r_vec, jnp.full((16,), c, jnp.int32)], x)

    for j in range(NBUF):
        gather(j, j)

    @pl.loop(0, n_chunks - NBUF, step=NBUF)
    def _(g0):
        for j in range(NBUF):
            g = g0 + j
            wait_gather(g, j)
            compact(g, j)
            put(g, j)
            wait_put(g, j)
            gather(g + NBUF, j)

    for j in range(NBUF):
        g = n_chunks - NBUF + j
        wait_gather(g, j)
        compact(g, j)
        put(g, j)
    for j in range(NBUF):
        wait_put(n_chunks - NBUF + j, j)


def _mask_body(idx_ref, out_ref):
    out_ref[...] = idx_ref[...] == PAD_ID


@jax.jit
def kernel(word_indices, vocabulary):
    n_rows, seq = word_indices.shape
    b = n_rows * seq
    bpw = b // NW
    chunk = 128
    n_chunks = bpw // chunk

    t2 = vocabulary.reshape(500000, 128)
    idx_flat = word_indices.reshape(NW, n_chunks, chunk)

    mesh = plsc.VectorSubcoreMesh(core_axis_name="c", subcore_axis_name="s")
    gathered = pl.kernel(
        functools.partial(_gather_body, n_chunks, chunk),
        out_type=jax.ShapeDtypeStruct((b, D), jnp.float32),
        mesh=mesh,
        scratch_types=[
            pltpu.VMEM((n_chunks, chunk), jnp.int32),
            pltpu.VMEM((n_chunks, chunk), jnp.int32),
            tuple(pltpu.VMEM((chunk, 128), jnp.float32) for _ in range(NBUF)),
            tuple(pltpu.VMEM((chunk, D), jnp.float32) for _ in range(NBUF)),
            tuple(pltpu.SemaphoreType.DMA for _ in range(NBUF)),
            tuple(pltpu.SemaphoreType.DMA for _ in range(NBUF)),
        ],
        compiler_params=pltpu.CompilerParams(use_tc_tiling_on_sc=True,
                                             needs_layout_passes=False),
    )(idx_flat, t2)

    mask = pl.pallas_call(
        _mask_body,
        out_shape=jax.ShapeDtypeStruct((n_rows, seq), jnp.bool_),
    )(word_indices)

    return gathered.reshape(n_rows, seq, D), mask


# speedup vs baseline: 1.5782x; 1.5782x over previous
import functools

import jax
import jax.numpy as jnp
from jax import lax
from jax.experimental import pallas as pl
from jax.experimental.pallas import tpu as pltpu
from jax.experimental.pallas import tpu_sc as plsc

PAD_ID = 999999
D = 64
NC = 2
NS = 16
NW = NC * NS

VOC = 1000000
ABLK = 256
NA_BLOCKS = VOC // ABLK
CHUNK = 40
NBUF = 2


def _prep_body(idx_ref, idx2_ref, off_ref, mask_ref):
    idx = idx_ref[...]
    idx2_ref[...] = idx >> 1
    off_ref[...] = (idx & 1) * D
    mask_ref[...] = idx == PAD_ID


def _detile_body(voc_hbm, t2_hbm, bufa, bufb, rsems, wsems):
    wid = lax.axis_index("s") * NC + lax.axis_index("c")
    nb = ABLK // 2

    def src_ref(c, n=ABLK):
        return voc_hbm.at[pl.ds(c * ABLK, n)]

    def dst_ref(c, n=nb):
        return t2_hbm.at[pl.ds(c * nb, n)]

    def read(c, j):
        return pltpu.async_copy(src_ref(c), bufa[j], rsems[j])

    def wait_read(c, j):
        pltpu.make_async_copy(src_ref(c), bufa[j], rsems[j]).wait()

    def write(c, j):
        return pltpu.async_copy(bufb[j], dst_ref(c), wsems[j])

    def wait_write(c, j):
        pltpu.make_async_copy(bufb[j], dst_ref(c), wsems[j]).wait()

    def repack(j):
        @pl.loop(0, nb)
        def _(p):
            for k in range(D // 16):
                bufb[j][p, pl.ds(k * 16, 16)] = bufa[j][2 * p,
                                                        pl.ds(k * 16, 16)]
                bufb[j][p, pl.ds(D + k * 16, 16)] = bufa[j][2 * p + 1,
                                                            pl.ds(k * 16, 16)]

    n_rounds = (NA_BLOCKS + NW - 1) // NW

    for j in range(2):
        c0 = j * NW + wid

        @pl.when(c0 < NA_BLOCKS)
        def _():
            read(c0, j)

    @pl.loop(0, n_rounds)
    def _(k):
        for j in range(2):
            c = (2 * k + j) * NW + wid

            @pl.when(c < NA_BLOCKS)
            def _():
                wait_read(c, j)
                repack(j)
                write(c, j)
                wait_write(c, j)
                nxt = (2 * k + j + 2) * NW + wid

                @pl.when(nxt < NA_BLOCKS)
                def _():
                    read(nxt, j)

    @pl.when(wid == 0)
    def _():
        tail = NA_BLOCKS * ABLK
        src = voc_hbm.at[pl.ds(tail, 64)]
        bsrc = bufa[0].at[pl.ds(0, 64)]
        pltpu.async_copy(src, bsrc, rsems[0])
        pltpu.make_async_copy(src, bsrc, rsems[0]).wait()

        @pl.loop(0, 32)
        def _(p):
            for k in range(D // 16):
                bufb[0][p, pl.ds(k * 16, 16)] = bufa[0][2 * p,
                                                        pl.ds(k * 16, 16)]
                bufb[0][p, pl.ds(D + k * 16, 16)] = bufa[0][2 * p + 1,
                                                            pl.ds(k * 16, 16)]
        bdst = bufb[0].at[pl.ds(0, 32)]
        dst = t2_hbm.at[pl.ds(tail // 2, 32)]
        pltpu.async_copy(bdst, dst, wsems[0])
        pltpu.make_async_copy(bdst, dst, wsems[0]).wait()


def _gather_body(n_chunks, idx2_hbm, off_hbm, t2_hbm, out_hbm,
                 idx2_v, off_v, rows, comp, gsems, osems):
    wid = lax.axis_index("s") * NC + lax.axis_index("c")
    sent0 = wid * 128
    cps = 200 // CHUNK

    pltpu.sync_copy(idx2_hbm.at[wid], idx2_v)
    pltpu.sync_copy(off_hbm.at[wid], off_v)

    def gather(g, j):
        return pltpu.async_copy(
            t2_hbm.at[idx2_v.at[pl.ds(g * CHUNK, CHUNK)]], rows[j], gsems[j])

    def out_slice(g):
        return out_hbm.at[sent0 + g // cps, pl.ds((g % cps) * CHUNK, CHUNK)]

    def put(g, j):
        return pltpu.async_copy(comp[j], out_slice(g), osems[j])

    def wait_gather(g, j):
        pltpu.make_async_copy(
            t2_hbm.at[idx2_v.at[pl.ds(g * CHUNK, CHUNK)]], rows[j],
            gsems[j]).wait()

    def wait_put(g, j):
        pltpu.make_async_copy(comp[j], out_slice(g), osems[j]).wait()

    def compact(g, j):
        ovs = [off_v[pl.ds(g * CHUNK + r0, 16)] for r0 in (0, 16, 24)]
        for r in range(CHUNK):
            if r < 32:
                off = ovs[r // 16][r % 16]
            else:
                off = ovs[2][r - 24]
            for k in range(D // 16):
                comp[j][r, pl.ds(k * 16, 16)] = (
                    rows[j][r, pl.ds(off + k * 16, 16)])

    for j in range(NBUF):
        gather(j, j)

    @pl.loop(0, n_chunks - NBUF, step=NBUF)
    def _(g0):
        for j in range(NBUF):
            g = g0 + j
            wait_gather(g, j)
            compact(g, j)
            put(g, j)
            wait_put(g, j)
            gather(g + NBUF, j)

    for j in range(NBUF):
        g = n_chunks - NBUF + j
        wait_gather(g, j)
        compact(g, j)
        put(g, j)
    for j in range(NBUF):
        wait_put(n_chunks - NBUF + j, j)


@jax.jit
def kernel(word_indices, vocabulary):
    n_rows, seq = word_indices.shape
    b = n_rows * seq
    n_chunks = b // (NW * CHUNK)

    mesh = plsc.VectorSubcoreMesh(core_axis_name="c", subcore_axis_name="s")
    sc_params = pltpu.CompilerParams(use_tc_tiling_on_sc=True,
                                     needs_layout_passes=False)

    idx2, off, mask = pl.pallas_call(
        _prep_body,
        out_shape=(
            jax.ShapeDtypeStruct((n_rows, seq), jnp.int32),
            jax.ShapeDtypeStruct((n_rows, seq), jnp.int32),
            jax.ShapeDtypeStruct((n_rows, seq), jnp.bool_),
        ),
    )(word_indices)

    t2 = pl.kernel(
        _detile_body,
        out_type=jax.ShapeDtypeStruct((VOC // 2, 128), jnp.float32),
        mesh=mesh,
        scratch_types=[
            tuple(pltpu.VMEM((ABLK, D), jnp.float32) for _ in range(2)),
            tuple(pltpu.VMEM((ABLK // 2, 128), jnp.float32)
                  for _ in range(2)),
            tuple(pltpu.SemaphoreType.DMA for _ in range(2)),
            tuple(pltpu.SemaphoreType.DMA for _ in range(2)),
        ],
        compiler_params=sc_params,
    )(vocabulary)

    embedded = pl.kernel(
        functools.partial(_gather_body, n_chunks),
        out_type=jax.ShapeDtypeStruct((n_rows, seq, D), jnp.float32),
        mesh=mesh,
        scratch_types=[
            pltpu.VMEM((n_chunks * CHUNK,), jnp.int32),
            pltpu.VMEM((n_chunks * CHUNK,), jnp.int32),
            tuple(pltpu.VMEM((CHUNK, 128), jnp.float32) for _ in range(NBUF)),
            tuple(pltpu.VMEM((CHUNK, D), jnp.float32) for _ in range(NBUF)),
            tuple(pltpu.SemaphoreType.DMA for _ in range(NBUF)),
            tuple(pltpu.SemaphoreType.DMA for _ in range(NBUF)),
        ],
        compiler_params=sc_params,
    )(idx2.reshape(NW, n_chunks * CHUNK), off.reshape(NW, n_chunks * CHUNK),
      t2)

    return embedded, mask


# speedup vs baseline: 2.2466x; 1.4235x over previous
import functools

import jax
import jax.numpy as jnp
from jax import lax
from jax.experimental import pallas as pl
from jax.experimental.pallas import tpu as pltpu
from jax.experimental.pallas import tpu_sc as plsc

PAD_ID = 999999
D = 64

NC = 2
NS = 16
NW = NC * NS

NBUF = 4
NUM_CORES = 2


def _gather_body(n_chunks, chunk, idx_hbm, table_hbm, out_hbm,
                 idx_v, rows, gsems, osems):
    wid = lax.axis_index("s") * NUM_CORES + lax.axis_index("c")
    bpw = n_chunks * chunk
    base = wid * bpw

    pltpu.sync_copy(idx_hbm.at[wid], idx_v)

    def gather(g, j):
        return pltpu.async_copy(table_hbm.at[idx_v.at[g]], rows[j], gsems[j])

    def out_slice(g):
        return out_hbm.at[pl.ds(base + g * chunk, chunk)]

    def put(g, j):
        return pltpu.async_copy(rows[j], out_slice(g), osems[j])

    def wait_gather(g, j):
        pltpu.make_async_copy(table_hbm.at[idx_v.at[g]], rows[j],
                              gsems[j]).wait()

    def wait_put(g, j):
        pltpu.make_async_copy(rows[j], out_slice(g), osems[j]).wait()

    for j in range(NBUF):
        gather(j, j)

    @pl.loop(0, n_chunks - NBUF, step=NBUF)
    def _(g0):
        for j in range(NBUF):
            g = g0 + j
            wait_gather(g, j)
            put(g, j)
            wait_put(g, j)
            gather(g + NBUF, j)

    for j in range(NBUF):
        g = n_chunks - NBUF + j
        wait_gather(g, j)
        put(g, j)
    for j in range(NBUF):
        wait_put(n_chunks - NBUF + j, j)


def _mask_body(idx_ref, out_ref):
    out_ref[...] = idx_ref[...] == PAD_ID


@jax.jit
def kernel(word_indices, vocabulary):
    n_rows, seq = word_indices.shape
    b = n_rows * seq
    bpw = b // (NUM_CORES * NS)
    chunk = 128
    n_chunks = bpw // chunk

    idx_flat = word_indices.reshape(NUM_CORES * NS, n_chunks, chunk)

    mesh = plsc.VectorSubcoreMesh(core_axis_name="c", subcore_axis_name="s",
                                  num_cores=NUM_CORES)
    gathered = pl.kernel(
        functools.partial(_gather_body, n_chunks, chunk),
        out_type=jax.ShapeDtypeStruct((b, D), jnp.float32),
        mesh=mesh,
        scratch_types=[
            pltpu.VMEM((n_chunks, chunk), jnp.int32),
            tuple(pltpu.VMEM((chunk, D), jnp.float32) for _ in range(NBUF)),
            tuple(pltpu.SemaphoreType.DMA for _ in range(NBUF)),
            tuple(pltpu.SemaphoreType.DMA for _ in range(NBUF)),
        ],
        compiler_params=pltpu.CompilerParams(use_tc_tiling_on_sc=False,
                                             needs_layout_passes=False),
    )(idx_flat, vocabulary)

    mask = pl.pallas_call(
        _mask_body,
        out_shape=jax.ShapeDtypeStruct((n_rows, seq), jnp.bool_),
    )(word_indices)

    return gathered.reshape(n_rows, seq, D), mask
